# Initial kernel scaffold; baseline (speedup 1.0000x reference)
import jax
import jax.numpy as jnp
from jax.experimental import pallas as pl


def kernel(x, edge_index, batch, W1, b1, W2, b2, Wl, bl):
    raise NotImplementedError("write your pallas kernel here")


# speedup vs baseline: 10.8393x; 10.8393x over previous
import functools

import jax
import jax.numpy as jnp
from jax import lax
from jax.experimental import pallas as pl
from jax.experimental.pallas import tpu as pltpu
from jax.experimental.pallas import tpu_sc as plsc

N = 10000
E = 320000
D = 128
H = 128
C = 10
G = 64

NC, NS, L = 2, 16, 16
NW = NC * NS
NPAD = 10240
RPT = NPAD // NS
CHUNK = 128
NCHUNK = -(-E // (NW * CHUNK))
EPW = NCHUNK * CHUNK
EPAD = NW * EPW
PAD_ROW = N

_mesh = plsc.VectorSubcoreMesh(core_axis_name="c", subcore_axis_name="s",
                               num_cores=NC, num_subcores=NS)


@functools.partial(
    pl.kernel,
    out_type=jax.ShapeDtypeStruct((NC * NPAD,), jnp.float32),
    mesh=_mesh,
    scratch_types=[
        pltpu.VMEM((CHUNK,), jnp.int32),
        pltpu.VMEM((CHUNK,), jnp.float32),
        pltpu.VMEM((RPT,), jnp.float32),
        pltpu.VMEM_SHARED((NPAD,), jnp.float32),
    ],
)
def _deg_kernel(dst_hbm, out_hbm, dstv, onesv, zv, acc):
    cid = lax.axis_index("c")
    sid = lax.axis_index("s")
    wid = cid * NS + sid
    for j in range(CHUNK // L):
        onesv[pl.ds(j * L, L)] = jnp.ones((L,), jnp.float32)

    def zb(i, c):
        zv[pl.ds(i * L, L)] = jnp.zeros((L,), jnp.float32)
        return c

    lax.fori_loop(0, RPT // L, zb, 0)
    pltpu.sync_copy(zv, acc.at[pl.ds(sid * RPT, RPT)])
    plsc.subcore_barrier()

    base0 = wid * EPW

    def body(i, c):
        pltpu.sync_copy(dst_hbm.at[pl.ds(base0 + i * CHUNK, CHUNK)], dstv)
        pltpu.sync_copy(onesv, acc.at[dstv], add=True)
        return c

    lax.fori_loop(0, NCHUNK, body, 0)
    plsc.subcore_barrier()
    pltpu.sync_copy(acc.at[pl.ds(sid * RPT, RPT)],
                    out_hbm.at[pl.ds(cid * NPAD + sid * RPT, RPT)])


@functools.partial(
    pl.kernel,
    out_type=jax.ShapeDtypeStruct((NC * NPAD, H), jnp.float32),
    mesh=_mesh,
    scratch_types=[
        pltpu.VMEM((CHUNK,), jnp.int32),
        pltpu.VMEM((CHUNK,), jnp.int32),
        pltpu.VMEM((CHUNK, H), jnp.float32),
        pltpu.VMEM((64, H), jnp.float32),
        pltpu.SemaphoreType.DMA,
        pltpu.VMEM_SHARED((NPAD, H), jnp.float32),
    ],
)
def _edge_aggregate(h_hbm, src_hbm, dst_hbm, out_hbm,
                    srcv, dstv, rows, zrows, sem, acc):
    cid = lax.axis_index("c")
    sid = lax.axis_index("s")
    wid = cid * NS + sid

    def zb(i, c):
        for j in range(H // L):
            zrows[i, pl.ds(j * L, L)] = jnp.zeros((L,), jnp.float32)
        return c

    lax.fori_loop(0, 64, zb, 0)
    for t in range(RPT // 64):
        pltpu.sync_copy(zrows, acc.at[pl.ds(sid * RPT + t * 64, 64)])
    plsc.subcore_barrier()

    base0 = wid * EPW

    def body(i, c):
        b = base0 + i * CHUNK
        pltpu.sync_copy(src_hbm.at[pl.ds(b, CHUNK)], srcv)
        pltpu.sync_copy(dst_hbm.at[pl.ds(b, CHUNK)], dstv)
        pltpu.async_copy(h_hbm.at[srcv], rows, sem).wait()
        pltpu.sync_copy(rows, acc.at[dstv], add=True)
        return c

    lax.fori_loop(0, NCHUNK, body, 0)
    plsc.subcore_barrier()
    for t in range(RPT // 64):
        r0 = sid * RPT + t * 64
        pltpu.sync_copy(acc.at[pl.ds(r0, 64)],
                        out_hbm.at[pl.ds(cid * NPAD + r0, 64)])


def _stage1_body(d0, d1, x, w1, dinv_out, h1p_out):
    deg = d0[...] + d1[...] + 1.0
    dinv = lax.rsqrt(deg)
    dinv_out[...] = dinv
    h1p_out[...] = jnp.dot(x[...], w1[...],
                           preferred_element_type=jnp.float32) * dinv


_stage1 = pl.pallas_call(
    _stage1_body,
    out_shape=[jax.ShapeDtypeStruct((NPAD, 1), jnp.float32),
               jax.ShapeDtypeStruct((NPAD, H), jnp.float32)],
)


def _stage2_body(a0, a1, h1p, dinv, b1, w2, h2p_out):
    dv = dinv[...]
    z = (a0[...] + a1[...] + h1p[...]) * dv + b1[...]
    z = jnp.maximum(z, 0.0)
    h2p_out[...] = jnp.dot(z, w2[...],
                           preferred_element_type=jnp.float32) * dv


_stage2 = pl.pallas_call(
    _stage2_body,
    out_shape=jax.ShapeDtypeStruct((NPAD, H), jnp.float32),
)


def _stage3_body(a0, a1, h2p, dinv, b2, batch8, wl, bl, out):
    z = (a0[...] + a1[...] + h2p[...]) * dinv[...] + b2[...]
    ids = batch8[0:1, :]
    seg = lax.broadcasted_iota(jnp.int32, (G, NPAD), 0)
    oht = (seg == ids).astype(jnp.float32)
    sums = jnp.dot(oht, z, preferred_element_type=jnp.float32)
    counts = jnp.sum(oht, axis=1, keepdims=True)
    pooled = sums / jnp.maximum(counts, 1.0)
    out[...] = jnp.dot(pooled, wl[...],
                       preferred_element_type=jnp.float32) + bl[...]


_stage3 = pl.pallas_call(
    _stage3_body,
    out_shape=jax.ShapeDtypeStruct((G, C), jnp.float32),
)


def kernel(x, edge_index, batch, W1, b1, W2, b2, Wl, bl):
    f32 = jnp.float32
    src = jnp.full((EPAD,), PAD_ROW, jnp.int32).at[:E].set(edge_index[0])
    dst = jnp.full((EPAD,), PAD_ROW, jnp.int32).at[:E].set(edge_index[1])
    xp = jnp.zeros((NPAD, D), f32).at[:N].set(x)
    bpad = jnp.pad(batch.astype(jnp.int32), (0, NPAD - N), constant_values=G)
    batch8 = jnp.broadcast_to(bpad[None, :], (8, NPAD))

    degp = _deg_kernel(dst)
    d0 = degp[:NPAD].reshape(NPAD, 1)
    d1 = degp[NPAD:].reshape(NPAD, 1)

    dinv, h1p = _stage1(d0, d1, xp, W1)
    acc1 = _edge_aggregate(h1p, src, dst)
    h2p = _stage2(acc1[:NPAD], acc1[NPAD:], h1p, dinv,
                  b1.reshape(1, H), W2)
    acc2 = _edge_aggregate(h2p, src, dst)
    out = _stage3(acc2[:NPAD], acc2[NPAD:], h2p, dinv,
                  b2.reshape(1, H), batch8, Wl, bl.reshape(1, C))
    return out
